# baseline (speedup 1.0000x reference)
import functools

import jax
import jax.numpy as jnp
from jax import lax
from jax.experimental import pallas as pl
from jax.experimental.pallas import tpu as pltpu
from jax.experimental.pallas import tpu_sc as plsc

NC = 2
NS = 16
NW = NC * NS
CHUNK = 128
DEG_LANES = 16
NPAD = 10240
NBUF = 4


def _vector_mesh():
    return plsc.VectorSubcoreMesh(core_axis_name="c", subcore_axis_name="s")


def _sc_degree(dst, zeros_deg):
    e = dst.shape[0]
    n = zeros_deg.shape[0]
    t_chunks = e // (NW * CHUNK)
    rpt = n // NS

    @functools.partial(
        pl.kernel,
        out_type=jax.ShapeDtypeStruct((NC, n, DEG_LANES), jnp.float32),
        mesh=_vector_mesh(),
        scratch_types=[
            pltpu.VMEM((NBUF, CHUNK), jnp.int32),
            pltpu.VMEM((CHUNK, DEG_LANES), jnp.float32),
            pltpu.VMEM_SHARED((n, DEG_LANES), jnp.float32),
        ] + [pltpu.SemaphoreType.DMA] * (2 * NBUF),
    )
    def k(dst_hbm, zeros_hbm, out_hbm, idxv, ones_v, acc_sh, *sems):
        isem = sems[0:NBUF]
        ssem = sems[NBUF:2 * NBUF]
        core = lax.axis_index("c")
        sid = lax.axis_index("s")
        wid = sid * NC + core
        r0 = sid * rpt
        cbase = wid * t_chunks

        @pl.loop(0, CHUNK)
        def _(r):
            ones_v[r, :] = jnp.ones((DEG_LANES,), jnp.float32)

        pltpu.sync_copy(zeros_hbm.at[pl.ds(r0, rpt)], acc_sh.at[pl.ds(r0, rpt)])
        plsc.subcore_barrier()

        def issue_idx(c, b):
            pltpu.async_copy(dst_hbm.at[pl.ds((cbase + c) * CHUNK, CHUNK)],
                             idxv.at[b], isem[b])

        issue_idx(0, 0)
        issue_idx(1, 1)

        nslots = ((t_chunks + 2 + NBUF - 1) // NBUF) * NBUF

        @pl.loop(0, nslots, step=NBUF)
        def _(t):
            for j in range(NBUF):
                c = t + j
                jp2 = (j + 2) % NBUF

                @pl.when(c < t_chunks)
                def _():
                    pltpu.make_async_copy(
                        dst_hbm.at[pl.ds(0, CHUNK)], idxv.at[j], isem[j]).wait()
                    pltpu.async_copy(ones_v, acc_sh.at[idxv.at[j]],
                                     ssem[j], add=True)

                @pl.when(jnp.logical_and(c >= 2, c <= t_chunks + 1))
                def _():
                    pltpu.make_async_copy(
                        ones_v, acc_sh.at[idxv.at[jp2]], ssem[jp2]).wait()

                @pl.when(c + 2 < t_chunks)
                def _():
                    issue_idx(c + 2, jp2)

        plsc.subcore_barrier()
        pltpu.sync_copy(acc_sh.at[pl.ds(r0, rpt)],
                        out_hbm.at[core, pl.ds(r0, rpt)])

    return k(dst, zeros_deg)


def _sc_aggregate(g, src, dst, zeros_nd):
    d = g.shape[1]
    n = zeros_nd.shape[0]
    e = src.shape[0]
    t_chunks = e // (NW * CHUNK)
    rpt = n // NS

    @functools.partial(
        pl.kernel,
        out_type=jax.ShapeDtypeStruct((NC, n, d), jnp.float32),
        mesh=_vector_mesh(),
        scratch_types=[
            pltpu.VMEM((NBUF, CHUNK), jnp.int32),
            pltpu.VMEM((NBUF, CHUNK), jnp.int32),
            pltpu.VMEM((2, CHUNK, d), jnp.float32),
            pltpu.VMEM_SHARED((n, d), jnp.float32),
        ] + [pltpu.SemaphoreType.DMA] * (2 * NBUF),
    )
    def k(g_hbm, src_hbm, dst_hbm, zeros_hbm, out_hbm,
          srcv, dstv, rows, acc_sh, *sems):
        isem = sems[0:NBUF]
        gsem = sems[NBUF:NBUF + 2]
        ssem = sems[NBUF + 2:NBUF + 4]
        core = lax.axis_index("c")
        sid = lax.axis_index("s")
        wid = sid * NC + core
        r0 = sid * rpt
        cbase = wid * t_chunks

        pltpu.sync_copy(zeros_hbm.at[pl.ds(r0, rpt)], acc_sh.at[pl.ds(r0, rpt)])
        plsc.subcore_barrier()

        def issue_idx(c, b):
            base = (cbase + c) * CHUNK
            pltpu.async_copy(src_hbm.at[pl.ds(base, CHUNK)], srcv.at[b], isem[b])
            pltpu.async_copy(dst_hbm.at[pl.ds(base, CHUNK)], dstv.at[b], isem[b])

        def wait_idx(b):
            pltpu.make_async_copy(src_hbm.at[pl.ds(0, CHUNK)], srcv.at[b], isem[b]).wait()
            pltpu.make_async_copy(dst_hbm.at[pl.ds(0, CHUNK)], dstv.at[b], isem[b]).wait()

        issue_idx(0, 0)
        issue_idx(1, 1)

        nslots = ((t_chunks + 2 + NBUF - 1) // NBUF) * NBUF

        @pl.loop(0, nslots, step=NBUF)
        def _(t):
            for j in range(NBUF):
                c = t + j
                r = j % 2
                rm1 = (j - 1) % 2
                im1 = (j - 1) % NBUF
                ip2 = (j + 2) % NBUF

                @pl.when(jnp.logical_and(c >= 2, c <= t_chunks + 1))
                def _():
                    pltpu.make_async_copy(
                        rows.at[r], acc_sh.at[dstv.at[ip2]], ssem[r]).wait()

                @pl.when(c < t_chunks)
                def _():
                    wait_idx(j)
                    pltpu.async_copy(g_hbm.at[srcv.at[j]], rows.at[r], gsem[r])

                @pl.when(jnp.logical_and(c >= 1, c <= t_chunks))
                def _():
                    pltpu.make_async_copy(
                        g_hbm.at[srcv.at[im1]], rows.at[rm1], gsem[rm1]).wait()
                    pltpu.async_copy(rows.at[rm1], acc_sh.at[dstv.at[im1]],
                                     ssem[rm1], add=True)

                @pl.when(c + 2 < t_chunks)
                def _():
                    issue_idx(c + 2, ip2)

        plsc.subcore_barrier()
        pltpu.sync_copy(acc_sh.at[pl.ds(r0, rpt)],
                        out_hbm.at[core, pl.ds(r0, rpt)])

    return k(g, src, dst, zeros_nd)


_BM = 2000


def _tc_matmul(x, w):
    n, d = x.shape

    def body(x_ref, w_ref, o_ref):
        o_ref[...] = lax.dot_general(
            x_ref[...], w_ref[...], (((1,), (1,)), ((), ())),
            preferred_element_type=jnp.float32)

    return pl.pallas_call(
        body,
        grid=(n // _BM,),
        in_specs=[pl.BlockSpec((_BM, d), lambda i: (i, 0)),
                  pl.BlockSpec((d, d), lambda i: (0, 0))],
        out_specs=pl.BlockSpec((_BM, d), lambda i: (i, 0)),
        out_shape=jax.ShapeDtypeStruct((n, d), jnp.float32),
    )(x, w)


def _dinv_from(dg_ref):
    deg = dg_ref[0, :, 0] + dg_ref[1, :, 0] + 1.0
    return lax.rsqrt(deg)


def _tc_scale(h, degp):
    n, d = h.shape

    def body(h_ref, dg_ref, o_ref):
        dinv = _dinv_from(dg_ref)
        o_ref[...] = h_ref[...] * dinv[:, None]

    return pl.pallas_call(
        body,
        grid=(n // _BM,),
        in_specs=[pl.BlockSpec((_BM, d), lambda i: (i, 0)),
                  pl.BlockSpec((NC, _BM, DEG_LANES), lambda i: (0, i, 0))],
        out_specs=pl.BlockSpec((_BM, d), lambda i: (i, 0)),
        out_shape=jax.ShapeDtypeStruct((n, d), jnp.float32),
    )(h, degp)


def _tc_mid(g1, accp, degp, w2, b1):
    n, d = g1.shape

    def body(g_ref, a_ref, dg_ref, w_ref, b_ref, o_ref):
        dinv = _dinv_from(dg_ref)
        s = (a_ref[0] + a_ref[1] + g_ref[...]) * dinv[:, None] + b_ref[...]
        h1 = jnp.maximum(s, 0.0)
        o_ref[...] = lax.dot_general(
            h1, w_ref[...], (((1,), (1,)), ((), ())),
            preferred_element_type=jnp.float32) * dinv[:, None]

    return pl.pallas_call(
        body,
        grid=(n // _BM,),
        in_specs=[pl.BlockSpec((_BM, d), lambda i: (i, 0)),
                  pl.BlockSpec((NC, _BM, d), lambda i: (0, i, 0)),
                  pl.BlockSpec((NC, _BM, DEG_LANES), lambda i: (0, i, 0)),
                  pl.BlockSpec((d, d), lambda i: (0, 0)),
                  pl.BlockSpec((1, d), lambda i: (0, 0))],
        out_specs=pl.BlockSpec((_BM, d), lambda i: (i, 0)),
        out_shape=jax.ShapeDtypeStruct((n, d), jnp.float32),
    )(g1, accp, degp, w2, b1)


def _tc_final(g2, accp, degp, b2):
    n, d = g2.shape

    def body(g_ref, a_ref, dg_ref, b_ref, o_ref):
        dinv = _dinv_from(dg_ref)
        s = (a_ref[0] + a_ref[1] + g_ref[...]) * dinv[:, None] + b_ref[...]
        o_ref[...] = jax.nn.sigmoid(s)

    return pl.pallas_call(
        body,
        grid=(n // _BM,),
        in_specs=[pl.BlockSpec((_BM, d), lambda i: (i, 0)),
                  pl.BlockSpec((NC, _BM, d), lambda i: (0, i, 0)),
                  pl.BlockSpec((NC, _BM, DEG_LANES), lambda i: (0, i, 0)),
                  pl.BlockSpec((1, d), lambda i: (0, 0))],
        out_specs=pl.BlockSpec((_BM, d), lambda i: (i, 0)),
        out_shape=jax.ShapeDtypeStruct((n, d), jnp.float32),
    )(g2, accp, degp, b2)


def kernel(x, edge_index, embed, W1, b1, W2, b2):
    n, d = embed.shape
    ei = edge_index.astype(jnp.int32)
    e = ei.shape[1]
    e_pad = ((e + NW * CHUNK - 1) // (NW * CHUNK)) * (NW * CHUNK)
    src = jnp.concatenate([ei[0], jnp.zeros((e_pad - e,), jnp.int32)])
    dst = jnp.concatenate([ei[1], jnp.full((e_pad - e,), NPAD - 8, jnp.int32)])
    zeros_nd = jnp.zeros((NPAD, d), jnp.float32)
    zeros_deg = jnp.zeros((NPAD, DEG_LANES), jnp.float32)

    degp = _sc_degree(dst, zeros_deg)
    h_lin = _tc_matmul(embed, W1)
    g1 = _tc_scale(h_lin, degp)
    acc1 = _sc_aggregate(g1, src, dst, zeros_nd)
    g2 = _tc_mid(g1, acc1, degp, W2, b1.reshape(1, d))
    acc2 = _sc_aggregate(g2, src, dst, zeros_nd)
    return _tc_final(g2, acc2, degp, b2.reshape(1, d))
